# baseline (speedup 1.0000x reference)
import functools

import jax
import jax.numpy as jnp
from jax import lax
from jax.experimental import pallas as pl
from jax.experimental.pallas import tpu as pltpu
from jax.experimental.pallas import tpu_sc as plsc

NUM_CORES = 2
NUM_SUBCORES = 16
LANES = 16
NW = NUM_CORES * NUM_SUBCORES
R = 5

NEG_INF = float("-inf")
POS_INF = float("inf")


def _make_kernel(B, L):
    rows_per = B // NW
    mesh = plsc.VectorSubcoreMesh(
        core_axis_name="c", subcore_axis_name="s",
        num_cores=NUM_CORES, num_subcores=NUM_SUBCORES)

    @functools.partial(
        pl.kernel,
        out_type=jax.ShapeDtypeStruct((B, LANES), jnp.float32),
        mesh=mesh,
        compiler_params=pltpu.CompilerParams(needs_layout_passes=False),
        scratch_types=[
            pltpu.VMEM((rows_per, L), jnp.float32),
            pltpu.VMEM((rows_per, LANES), jnp.float32),
            pltpu.VMEM((rows_per,), jnp.int32),
        ],
    )
    def k(x_hbm, len_hbm, out_hbm, xbuf, obuf, lenbuf):
        wid = lax.axis_index("s") * NUM_CORES + lax.axis_index("c")
        base = wid * rows_per
        pltpu.sync_copy(len_hbm.at[pl.ds(base, rows_per)], lenbuf)
        pltpu.sync_copy(x_hbm.at[pl.ds(base, rows_per)], xbuf)

        iota = lax.iota(jnp.int32, LANES)
        minf = jnp.full((LANES,), NEG_INF, jnp.float32)
        pinf = jnp.full((LANES,), POS_INF, jnp.float32)

        def row_body(r, _):
            lv = lenbuf[pl.ds((r // LANES) * LANES, LANES)]
            lf = jnp.where(iota == r % LANES, lv.astype(jnp.float32), 0.0)
            l = jnp.max(lf, axis=0).astype(jnp.int32)
            out_vec = pinf
            for i in range(R):
                s = (i * l) // R
                e = ((i + 1) * l + (R - 1)) // R
                v0 = s // LANES
                v1 = (e + LANES - 1) // LANES

                def wbody(v, carry):
                    amax, amin = carry
                    x = xbuf[r, pl.ds(v * LANES, LANES)]
                    idx = v * LANES + iota
                    m = (idx >= s) & (idx < e)
                    amax = jnp.maximum(amax, jnp.where(m, x, minf))
                    amin = jnp.minimum(amin, jnp.where(m, x, pinf))
                    return amax, amin

                amax, amin = lax.fori_loop(v0, v1, wbody, (minf, pinf))
                mx = jnp.max(amax, axis=0)
                mn = jnp.min(amin, axis=0)
                out_vec = jnp.where(iota == i, mx, out_vec)
                out_vec = jnp.where(iota == R + i, mn, out_vec)
            obuf[r, :] = lax.sort(out_vec)
            return 0

        lax.fori_loop(0, rows_per, row_body, 0)
        pltpu.sync_copy(obuf, out_hbm.at[pl.ds(base, rows_per)])

    return k


@jax.jit
def kernel(inputs, lengths):
    B, L = inputs.shape
    out16 = _make_kernel(B, L)(inputs, lengths.astype(jnp.int32))
    return out16[:, : 2 * R]
